# baseline (speedup 1.0000x reference)
import functools

import jax
import jax.numpy as jnp
from jax import lax
from jax.experimental import pallas as pl
from jax.experimental.pallas import tpu as pltpu
from jax.experimental.pallas import tpu_sc as plsc

N = 10000
E = 320000
D = 128
H = 128
NPAD = 10240
NROW = NPAD // H
NTILES = 32
EPT = E // NTILES
C = 40
NCHUNK = EPT // C
G = 10
NG = NCHUNK // G
RPT = NPAD // 16


def _msg_body(x_ref, g_ref, b_ref, w_ref, bm_ref, o_ref):
    h = x_ref[...] * g_ref[...] + b_ref[...]
    o_ref[...] = jax.nn.gelu(
        jnp.dot(h, w_ref[...], preferred_element_type=jnp.float32)
        + bm_ref[...])


def _node_messages(x, g1, b1, w, bm):
    blk = 1000
    return pl.pallas_call(
        _msg_body,
        grid=(N // blk,),
        in_specs=[
            pl.BlockSpec((blk, D), lambda i: (i, 0)),
            pl.BlockSpec((1, D), lambda i: (0, 0)),
            pl.BlockSpec((1, D), lambda i: (0, 0)),
            pl.BlockSpec((D, H), lambda i: (0, 0)),
            pl.BlockSpec((1, H), lambda i: (0, 0)),
        ],
        out_specs=pl.BlockSpec((blk, H), lambda i: (i, 0)),
        out_shape=jax.ShapeDtypeStruct((N, H), jnp.float32),
    )(x, g1, b1, w, bm)


def _edge_body(at_ref, w_ref, be_ref, o_ref):
    m = lax.dot_general(at_ref[...], w_ref[...],
                        (((0,), (0,)), ((), ())),
                        preferred_element_type=jnp.float32)
    o_ref[...] = jax.nn.gelu(m + be_ref[...])


def _edge_messages(edge_attr_t, w, be):
    blk = 12800
    de = edge_attr_t.shape[0]
    return pl.pallas_call(
        _edge_body,
        grid=(E // blk,),
        in_specs=[
            pl.BlockSpec((de, blk), lambda i: (0, i)),
            pl.BlockSpec((de, H), lambda i: (0, 0)),
            pl.BlockSpec((1, H), lambda i: (0, 0)),
        ],
        out_specs=pl.BlockSpec((blk, H), lambda i: (i, 0)),
        out_shape=jax.ShapeDtypeStruct((E, H), jnp.float32),
    )(edge_attr_t, w, be)


_SC_MESH = plsc.VectorSubcoreMesh(core_axis_name="c", subcore_axis_name="s")


@functools.partial(
    pl.kernel,
    out_type=(jax.ShapeDtypeStruct((2, NPAD, H), jnp.float32),
              jax.ShapeDtypeStruct((NTILES, NROW, H), jnp.float32)),
    mesh=_SC_MESH,
    compiler_params=pltpu.CompilerParams(use_tc_tiling_on_sc=False,
                                         needs_layout_passes=False),
    scratch_types=[
        pltpu.VMEM((2, G, C), jnp.int32),
        pltpu.VMEM((2, G, C), jnp.int32),
        pltpu.VMEM((4, C, H), jnp.float32),
        pltpu.VMEM((3, C, H), jnp.float32),
        pltpu.VMEM((NROW, H), jnp.float32),
        pltpu.VMEM_SHARED((NPAD, H), jnp.float32),
        pltpu.SemaphoreType.DMA((2,)),
        pltpu.SemaphoreType.DMA((4,)),
        pltpu.SemaphoreType.DMA((3,)),
        pltpu.SemaphoreType.DMA((4,)),
    ],
)
def _sc_aggregate(y_hbm, er_hbm, src_hbm, dst_hbm, acc_hbm, hist_hbm,
                  src_v, dst_v, rows_v, er_v, hist_v, acc,
                  sem_i, sem_g, sem_e, sem_s):
    cid = lax.axis_index("c")
    sid = lax.axis_index("s")
    wid = cid * 16 + sid
    ebase = pl.multiple_of(wid * EPT, 8)

    zero16 = jnp.zeros((16,), jnp.float32)
    ones16 = jnp.ones((16,), jnp.float32)

    def _hzero(i, carry):
        for h in range(H // 16):
            hist_v[i, pl.ds(h * 16, 16)] = zero16
        return carry

    lax.fori_loop(0, NROW, _hzero, 0)

    def _ezero(i, carry):
        for h in range(H // 16):
            er_v[0, i, pl.ds(h * 16, 16)] = zero16
        return carry

    lax.fori_loop(0, C, _ezero, 0)

    def _zcopy(j, carry):
        pltpu.sync_copy(er_v.at[0], acc.at[pl.ds(sid * RPT + j * C, C)])
        return carry

    lax.fori_loop(0, RPT // C, _zcopy, 0)
    plsc.subcore_barrier()

    def issue_idxgrp(g, gslot):
        row = wid * NCHUNK + g * G
        pltpu.async_copy(src_hbm.at[pl.ds(row, G)], src_v.at[gslot],
                         sem_i.at[gslot])
        pltpu.async_copy(dst_hbm.at[pl.ds(row, G)], dst_v.at[gslot],
                         sem_i.at[gslot])

    def wait_idxgrp(gslot):
        pltpu.make_async_copy(src_hbm.at[pl.ds(0, G)], src_v.at[gslot],
                              sem_i.at[gslot]).wait()
        pltpu.make_async_copy(dst_hbm.at[pl.ds(0, G)], dst_v.at[gslot],
                              sem_i.at[gslot]).wait()

    def issue_fetch(gslot, k, r, j):
        off = pl.multiple_of(ebase + j * C, 8)
        pltpu.async_copy(y_hbm.at[src_v.at[gslot, k]], rows_v.at[r],
                         sem_g.at[r])
        pltpu.async_copy(er_hbm.at[pl.ds(off, C)], er_v.at[lax.rem(j, 3)],
                         sem_e.at[lax.rem(j, 3)])

    def wait_fetch(gslot, k, r, j):
        pltpu.make_async_copy(y_hbm.at[src_v.at[gslot, k]], rows_v.at[r],
                              sem_g.at[r]).wait()
        pltpu.make_async_copy(er_hbm.at[pl.ds(0, C)],
                              er_v.at[lax.rem(j, 3)],
                              sem_e.at[lax.rem(j, 3)]).wait()

    def issue_scatter(gslot, k, r):
        pltpu.async_copy(rows_v.at[r], acc.at[dst_v.at[gslot, k]],
                         sem_s.at[r], add=True)

    def wait_scatter(gslot, k, r):
        pltpu.make_async_copy(rows_v.at[r], acc.at[dst_v.at[gslot, k]],
                              sem_s.at[r]).wait()

    def slots(j):
        g = lax.div(j, G)
        return lax.rem(g, 2), lax.rem(j, G), lax.rem(j, 4)

    issue_idxgrp(0, 0)
    wait_idxgrp(0)
    issue_idxgrp(1, 1)
    issue_fetch(0, 0, 0, 0)
    issue_fetch(0, 1, 1, 1)

    tail_mask = lax.iota(jnp.int32, 16) >= (16 - C % 16)

    def _chunk(j, carry):
        gs, k, r = slots(j)
        g = lax.div(j, G)
        wait_fetch(gs, k, r, j)

        @pl.when(j >= 2)
        def _drain_prev():
            g2, k2, r2 = slots(j - 2)
            wait_scatter(g2, k2, r2)

        @pl.when(j < NCHUNK - 2)
        def _prefetch():
            gn, kn, rn = slots(j + 2)

            @pl.when(kn == 0)
            def _w():
                wait_idxgrp(gn)

            issue_fetch(gn, kn, rn, j + 2)

        @pl.when(jnp.logical_and(k == 2,
                                 jnp.logical_and(g >= 1, g < NG - 1)))
        def _prefetch_idx():
            issue_idxgrp(g + 1, lax.rem(g + 1, 2))

        e2 = lax.rem(j, 3)

        def _mul(eo, c2):
            for u in range(4):
                e = eo * 4 + u
                for h in range(H // 16):
                    sl = pl.ds(h * 16, 16)
                    rows_v[r, e, sl] = rows_v[r, e, sl] * er_v[e2, e, sl]
            return c2

        lax.fori_loop(0, C // 4, _mul, 0)

        for i in range(C // 16):
            d16 = dst_v[gs, k, pl.ds(i * 16, 16)]
            hi = lax.shift_right_logical(d16, 7)
            lo = lax.bitwise_and(d16, 127)
            plsc.addupdate_scatter(hist_v, [hi, lo], ones16)
        if C % 16:
            d16 = dst_v[gs, k, pl.ds(C - 16, 16)]
            hi = lax.shift_right_logical(d16, 7)
            lo = lax.bitwise_and(d16, 127)
            plsc.addupdate_scatter(hist_v, [hi, lo], ones16, mask=tail_mask)

        issue_scatter(gs, k, r)
        return carry

    lax.fori_loop(0, NCHUNK, _chunk, 0)
    g2, k2, r2 = slots(NCHUNK - 2)
    wait_scatter(g2, k2, r2)
    g2, k2, r2 = slots(NCHUNK - 1)
    wait_scatter(g2, k2, r2)
    plsc.subcore_barrier()

    pltpu.sync_copy(acc.at[pl.ds(sid * RPT, RPT)],
                    acc_hbm.at[cid, pl.ds(sid * RPT, RPT)])
    pltpu.sync_copy(hist_v, hist_hbm.at[wid])


def _comb_body(acc_ref, hist_ref, x_ref, g2x_ref, b2x_ref, g2a_ref, b2a_ref,
               wx_ref, wa_ref, bu_ref, o_ref):
    blk = o_ref.shape[0]
    nrow = blk // H
    num = acc_ref[0] + acc_ref[1]
    cnt = jnp.sum(hist_ref[...], axis=0)
    cnt = jnp.maximum(cnt, 1.0)[:, :, None]
    agg = num.reshape(nrow, H, H) / cnt
    agg = agg.reshape(blk, H)
    hx = x_ref[...] * g2x_ref[...] + b2x_ref[...]
    ha = agg * g2a_ref[...] + b2a_ref[...]
    o_ref[...] = jax.nn.gelu(
        jnp.dot(hx, wx_ref[...], preferred_element_type=jnp.float32)
        + jnp.dot(ha, wa_ref[...], preferred_element_type=jnp.float32)
        + bu_ref[...])


def _combine(acc, hist, x, g2x, b2x, g2a, b2a, wx, wa, bu):
    blk = 1024
    nrow = blk // H
    return pl.pallas_call(
        _comb_body,
        grid=(NPAD // blk,),
        in_specs=[
            pl.BlockSpec((2, blk, H), lambda i: (0, i, 0)),
            pl.BlockSpec((NTILES, nrow, H), lambda i: (0, i, 0)),
            pl.BlockSpec((blk, D), lambda i: (i, 0)),
            pl.BlockSpec((1, D), lambda i: (0, 0)),
            pl.BlockSpec((1, D), lambda i: (0, 0)),
            pl.BlockSpec((1, H), lambda i: (0, 0)),
            pl.BlockSpec((1, H), lambda i: (0, 0)),
            pl.BlockSpec((D, H), lambda i: (0, 0)),
            pl.BlockSpec((H, H), lambda i: (0, 0)),
            pl.BlockSpec((1, H), lambda i: (0, 0)),
        ],
        out_specs=pl.BlockSpec((blk, H), lambda i: (i, 0)),
        out_shape=jax.ShapeDtypeStruct((NPAD, H), jnp.float32),
    )(acc, hist, x, g2x, b2x, g2a, b2a, wx, wa, bu)


def kernel(x, edge_index, edge_attr, gamma1, beta1, W_msg, b_msg,
           W_edge, b_edge, gamma2, beta2, W_upd, b_upd):
    dst = edge_index[0]
    src = edge_index[1]

    y = _node_messages(x, gamma1.reshape(1, D), beta1.reshape(1, D),
                       W_msg, b_msg.reshape(1, H))
    er = _edge_messages(edge_attr.T, W_edge, b_edge.reshape(1, H))
    acc, hist = _sc_aggregate(y, er, src.reshape(E // C, C),
                              dst.reshape(E // C, C))
    out = _combine(acc, hist, x,
                   gamma2[:D].reshape(1, D), beta2[:D].reshape(1, D),
                   gamma2[D:].reshape(1, H), beta2[D:].reshape(1, H),
                   W_upd[:D], W_upd[D:], b_upd.reshape(1, H))
    return out[:N]


# speedup vs baseline: 1.7735x; 1.7735x over previous
import functools

import jax
import jax.numpy as jnp
from jax import lax
from jax.experimental import pallas as pl
from jax.experimental.pallas import tpu as pltpu
from jax.experimental.pallas import tpu_sc as plsc

N = 10000
E = 320000
D = 128
H = 128
NPAD = 10240
NROW = NPAD // H
NTILES = 32
EPT = E // NTILES
C = 40
NCHUNK = EPT // C
G = 10
NG = NCHUNK // G
RPT = NPAD // 16


def _msg_body(x_ref, g_ref, b_ref, w_ref, bm_ref, o_ref):
    h = x_ref[...] * g_ref[...] + b_ref[...]
    o_ref[...] = jax.nn.gelu(
        jnp.dot(h, w_ref[...], preferred_element_type=jnp.float32)
        + bm_ref[...])


def _node_messages(x, g1, b1, w, bm):
    blk = 1000
    return pl.pallas_call(
        _msg_body,
        grid=(N // blk,),
        in_specs=[
            pl.BlockSpec((blk, D), lambda i: (i, 0)),
            pl.BlockSpec((1, D), lambda i: (0, 0)),
            pl.BlockSpec((1, D), lambda i: (0, 0)),
            pl.BlockSpec((D, H), lambda i: (0, 0)),
            pl.BlockSpec((1, H), lambda i: (0, 0)),
        ],
        out_specs=pl.BlockSpec((blk, H), lambda i: (i, 0)),
        out_shape=jax.ShapeDtypeStruct((N, H), jnp.float32),
    )(x, g1, b1, w, bm)


def _edge_body(at_ref, w_ref, be_ref, o_ref):
    m = lax.dot_general(at_ref[...], w_ref[...],
                        (((0,), (0,)), ((), ())),
                        preferred_element_type=jnp.float32)
    o_ref[...] = jax.nn.gelu(m + be_ref[...])


def _edge_messages(edge_attr_t, w, be):
    blk = 12800
    de = edge_attr_t.shape[0]
    return pl.pallas_call(
        _edge_body,
        grid=(E // blk,),
        in_specs=[
            pl.BlockSpec((de, blk), lambda i: (0, i)),
            pl.BlockSpec((de, H), lambda i: (0, 0)),
            pl.BlockSpec((1, H), lambda i: (0, 0)),
        ],
        out_specs=pl.BlockSpec((blk, H), lambda i: (i, 0)),
        out_shape=jax.ShapeDtypeStruct((E, H), jnp.float32),
    )(edge_attr_t, w, be)


_SC_MESH = plsc.VectorSubcoreMesh(core_axis_name="c", subcore_axis_name="s")


@functools.partial(
    pl.kernel,
    out_type=(jax.ShapeDtypeStruct((2, NPAD, H), jnp.float32),
              jax.ShapeDtypeStruct((NTILES, NROW, H), jnp.float32)),
    mesh=_SC_MESH,
    compiler_params=pltpu.CompilerParams(use_tc_tiling_on_sc=False,
                                         needs_layout_passes=False),
    scratch_types=[
        pltpu.VMEM((2, G, C), jnp.int32),
        pltpu.VMEM((2, G, C), jnp.int32),
        pltpu.VMEM((4, C, H), jnp.float32),
        pltpu.VMEM((3, C, H), jnp.float32),
        pltpu.VMEM((NROW, H), jnp.float32),
        pltpu.VMEM_SHARED((NPAD, H), jnp.float32),
        pltpu.SemaphoreType.DMA((2,)),
        pltpu.SemaphoreType.DMA((4,)),
        pltpu.SemaphoreType.DMA((3,)),
        pltpu.SemaphoreType.DMA((4,)),
    ],
)
def _sc_aggregate(y_hbm, er_hbm, src_hbm, dst_hbm, acc_hbm, hist_hbm,
                  src_v, dst_v, rows_v, er_v, hist_v, acc,
                  sem_i, sem_g, sem_e, sem_s):
    cid = lax.axis_index("c")
    sid = lax.axis_index("s")
    wid = cid * 16 + sid
    ebase = pl.multiple_of(wid * EPT, 8)

    zero16 = jnp.zeros((16,), jnp.float32)
    ones16 = jnp.ones((16,), jnp.float32)

    def _hzero(i, carry):
        for h in range(H // 16):
            hist_v[i, pl.ds(h * 16, 16)] = zero16
        return carry

    lax.fori_loop(0, NROW, _hzero, 0)

    def _ezero(i, carry):
        for h in range(H // 16):
            er_v[0, i, pl.ds(h * 16, 16)] = zero16
        return carry

    lax.fori_loop(0, C, _ezero, 0)

    def _zcopy(j, carry):
        pltpu.sync_copy(er_v.at[0], acc.at[pl.ds(sid * RPT + j * C, C)])
        return carry

    lax.fori_loop(0, RPT // C, _zcopy, 0)
    plsc.subcore_barrier()

    def issue_idxgrp(g, gslot):
        row = wid * NCHUNK + g * G
        pltpu.async_copy(src_hbm.at[pl.ds(row, G)], src_v.at[gslot],
                         sem_i.at[gslot])
        pltpu.async_copy(dst_hbm.at[pl.ds(row, G)], dst_v.at[gslot],
                         sem_i.at[gslot])

    def wait_idxgrp(gslot):
        pltpu.make_async_copy(src_hbm.at[pl.ds(0, G)], src_v.at[gslot],
                              sem_i.at[gslot]).wait()
        pltpu.make_async_copy(dst_hbm.at[pl.ds(0, G)], dst_v.at[gslot],
                              sem_i.at[gslot]).wait()

    def issue_fetch(gslot, k, r, j):
        off = pl.multiple_of(ebase + j * C, 8)
        pltpu.async_copy(y_hbm.at[src_v.at[gslot, k]], rows_v.at[r],
                         sem_g.at[r])
        pltpu.async_copy(er_hbm.at[pl.ds(off, C)], er_v.at[lax.rem(j, 3)],
                         sem_e.at[lax.rem(j, 3)])

    def wait_fetch(gslot, k, r, j):
        pltpu.make_async_copy(y_hbm.at[src_v.at[gslot, k]], rows_v.at[r],
                              sem_g.at[r]).wait()
        pltpu.make_async_copy(er_hbm.at[pl.ds(0, C)],
                              er_v.at[lax.rem(j, 3)],
                              sem_e.at[lax.rem(j, 3)]).wait()

    def issue_scatter(gslot, k, r):
        pltpu.async_copy(rows_v.at[r], acc.at[dst_v.at[gslot, k]],
                         sem_s.at[r], add=True)

    def wait_scatter(gslot, k, r):
        pltpu.make_async_copy(rows_v.at[r], acc.at[dst_v.at[gslot, k]],
                              sem_s.at[r]).wait()

    def slots(j):
        g = lax.div(j, G)
        return lax.rem(g, 2), lax.rem(j, G), lax.rem(j, 4)

    issue_idxgrp(0, 0)
    wait_idxgrp(0)
    issue_idxgrp(1, 1)
    issue_fetch(0, 0, 0, 0)
    issue_fetch(0, 1, 1, 1)

    tail_mask = lax.iota(jnp.int32, 16) >= (16 - C % 16)

    def _chunk(j, carry):
        gs, k, r = slots(j)
        g = lax.div(j, G)
        wait_fetch(gs, k, r, j)

        @pl.when(j >= 2)
        def _drain_prev():
            g2, k2, r2 = slots(j - 2)
            wait_scatter(g2, k2, r2)

        @pl.when(j < NCHUNK - 2)
        def _prefetch():
            gn, kn, rn = slots(j + 2)

            @pl.when(kn == 0)
            def _w():
                wait_idxgrp(gn)

            issue_fetch(gn, kn, rn, j + 2)

        @pl.when(jnp.logical_and(k == 2,
                                 jnp.logical_and(g >= 1, g < NG - 1)))
        def _prefetch_idx():
            issue_idxgrp(g + 1, lax.rem(g + 1, 2))

        e2 = lax.rem(j, 3)

        @plsc.parallel_loop(0, C, 1, unroll=4)
        def _mul(e):
            for h in range(H // 16):
                sl = pl.ds(h * 16, 16)
                rows_v[r, e, sl] = rows_v[r, e, sl] * er_v[e2, e, sl]

        for i in range(C // 16):
            d16 = dst_v[gs, k, pl.ds(i * 16, 16)]
            hi = lax.shift_right_logical(d16, 7)
            lo = lax.bitwise_and(d16, 127)
            plsc.addupdate_scatter(hist_v, [hi, lo], ones16)
        if C % 16:
            d16 = dst_v[gs, k, pl.ds(C - 16, 16)]
            hi = lax.shift_right_logical(d16, 7)
            lo = lax.bitwise_and(d16, 127)
            plsc.addupdate_scatter(hist_v, [hi, lo], ones16, mask=tail_mask)

        issue_scatter(gs, k, r)
        return carry

    lax.fori_loop(0, NCHUNK, _chunk, 0)
    g2, k2, r2 = slots(NCHUNK - 2)
    wait_scatter(g2, k2, r2)
    g2, k2, r2 = slots(NCHUNK - 1)
    wait_scatter(g2, k2, r2)
    plsc.subcore_barrier()

    pltpu.sync_copy(acc.at[pl.ds(sid * RPT, RPT)],
                    acc_hbm.at[cid, pl.ds(sid * RPT, RPT)])
    pltpu.sync_copy(hist_v, hist_hbm.at[wid])


def _comb_body(acc_ref, hist_ref, x_ref, g2x_ref, b2x_ref, g2a_ref, b2a_ref,
               wx_ref, wa_ref, bu_ref, o_ref):
    blk = o_ref.shape[0]
    nrow = blk // H
    num = acc_ref[0] + acc_ref[1]
    cnt = jnp.sum(hist_ref[...], axis=0)
    cnt = jnp.maximum(cnt, 1.0)[:, :, None]
    agg = num.reshape(nrow, H, H) / cnt
    agg = agg.reshape(blk, H)
    hx = x_ref[...] * g2x_ref[...] + b2x_ref[...]
    ha = agg * g2a_ref[...] + b2a_ref[...]
    o_ref[...] = jax.nn.gelu(
        jnp.dot(hx, wx_ref[...], preferred_element_type=jnp.float32)
        + jnp.dot(ha, wa_ref[...], preferred_element_type=jnp.float32)
        + bu_ref[...])


def _combine(acc, hist, x, g2x, b2x, g2a, b2a, wx, wa, bu):
    blk = 1024
    nrow = blk // H
    return pl.pallas_call(
        _comb_body,
        grid=(NPAD // blk,),
        in_specs=[
            pl.BlockSpec((2, blk, H), lambda i: (0, i, 0)),
            pl.BlockSpec((NTILES, nrow, H), lambda i: (0, i, 0)),
            pl.BlockSpec((blk, D), lambda i: (i, 0)),
            pl.BlockSpec((1, D), lambda i: (0, 0)),
            pl.BlockSpec((1, D), lambda i: (0, 0)),
            pl.BlockSpec((1, H), lambda i: (0, 0)),
            pl.BlockSpec((1, H), lambda i: (0, 0)),
            pl.BlockSpec((D, H), lambda i: (0, 0)),
            pl.BlockSpec((H, H), lambda i: (0, 0)),
            pl.BlockSpec((1, H), lambda i: (0, 0)),
        ],
        out_specs=pl.BlockSpec((blk, H), lambda i: (i, 0)),
        out_shape=jax.ShapeDtypeStruct((NPAD, H), jnp.float32),
    )(acc, hist, x, g2x, b2x, g2a, b2a, wx, wa, bu)


def kernel(x, edge_index, edge_attr, gamma1, beta1, W_msg, b_msg,
           W_edge, b_edge, gamma2, beta2, W_upd, b_upd):
    dst = edge_index[0]
    src = edge_index[1]

    y = _node_messages(x, gamma1.reshape(1, D), beta1.reshape(1, D),
                       W_msg, b_msg.reshape(1, H))
    er = _edge_messages(edge_attr.T, W_edge, b_edge.reshape(1, H))
    acc, hist = _sc_aggregate(y, er, src.reshape(E // C, C),
                              dst.reshape(E // C, C))
    out = _combine(acc, hist, x,
                   gamma2[:D].reshape(1, D), beta2[:D].reshape(1, D),
                   gamma2[D:].reshape(1, H), beta2[D:].reshape(1, H),
                   W_upd[:D], W_upd[D:], b_upd.reshape(1, H))
    return out[:N]
